# baseline (speedup 1.0000x reference)
import functools

import jax
import jax.numpy as jnp
from jax import lax
from jax.experimental import pallas as pl
from jax.experimental.pallas import tpu as pltpu
from jax.experimental.pallas import tpu_sc as plsc

N = 10000
F = 128
E = 320000
NC = 2
NS = 16
NW = NC * NS
L = 16
CB = 128
CPW = 80
CPW_FAST = 160
CPW_SLOW = 0
GB = 8
EW = CPW * CB
EPAD = EW * NW
NPAD = 10240
RPT = NPAD // NS


def _mesh():
    return plsc.VectorSubcoreMesh(
        core_axis_name="c", subcore_axis_name="s", num_cores=NC, num_subcores=NS
    )


def _fill_f32(ref, nrows, ncols, value):
    v = jnp.full((L,), value, jnp.float32)

    def body(i, carry):
        for j in range(ncols // L):
            ref[i, pl.ds(j * L, L)] = v
        return carry

    lax.fori_loop(0, nrows, body, 0)


def _make_seg(interpret=False):
    scratch = [
        pltpu.VMEM((GB, CB), jnp.int32),
        pltpu.VMEM((GB, CB), jnp.int32),
        pltpu.VMEM((CB, F), jnp.float32),
        pltpu.VMEM((CB, F), jnp.float32),
        pltpu.VMEM_SHARED((NPAD, F), jnp.float32),
        pltpu.SemaphoreType.DMA,
        pltpu.SemaphoreType.DMA,
        pltpu.SemaphoreType.DMA,
        pltpu.SemaphoreType.DMA,
    ]

    def body(h_hbm, srcs_hbm, dsts_hbm, out_hbm,
             src_v, dst_v, rows_a, rows_b, acc, sem_a, sem_b, sem_sa, sem_sb):
        cid = lax.axis_index("c")
        sid = lax.axis_index("s")
        base = jnp.where(cid == 1, sid * CPW_FAST,
                         NS * CPW_FAST + sid * CPW_SLOW)
        ng = jnp.where(cid == 1, CPW_FAST // GB, CPW_SLOW // GB)

        _fill_f32(rows_a, CB, F, 0.0)
        for i in range(RPT // CB):
            pltpu.sync_copy(rows_a, acc.at[pl.ds(sid * RPT + i * CB, CB)])
        plsc.subcore_barrier()

        bufs = ((rows_a, sem_a, sem_sa), (rows_b, sem_b, sem_sb))

        HCB = CB // 2

        def gath(k, buf, s1, s2):
            c1 = pltpu.async_copy(h_hbm.at[src_v.at[k, pl.ds(0, HCB)]],
                                  buf.at[pl.ds(0, HCB)], s1)
            c2 = pltpu.async_copy(h_hbm.at[src_v.at[k, pl.ds(HCB, HCB)]],
                                  buf.at[pl.ds(HCB, HCB)], s2)
            return (c1, c2)

        def gbody(g, carry):
            pltpu.sync_copy(srcs_hbm.at[pl.ds(base + g * GB, GB)], src_v)
            pltpu.sync_copy(dsts_hbm.at[pl.ds(base + g * GB, GB)], dst_v)
            cpg = [gath(0, rows_a, sem_a, sem_sa),
                   gath(1, rows_b, sem_b, sem_sb)]
            for k in range(GB):
                r, s1, s2 = bufs[k % 2]
                for c in cpg[k % 2]:
                    c.wait()
                pltpu.sync_copy(r, acc.at[dst_v.at[k]], add=True)
                if k + 2 < GB:
                    cpg[k % 2] = gath(k + 2, r, s1, s2)
            return carry

        lax.fori_loop(0, ng, gbody, 0)
        plsc.subcore_barrier()

        for i in range(RPT // CB):
            r0 = sid * RPT + i * CB
            pltpu.sync_copy(acc.at[pl.ds(r0, CB)], rows_a)
            pltpu.sync_copy(rows_a, out_hbm.at[pl.ds(cid * NPAD + r0, CB)])

    return pl.kernel(
        body,
        out_type=jax.ShapeDtypeStruct((NC * NPAD, F), jnp.float32),
        mesh=_mesh(),
        scratch_types=scratch,
        interpret=interpret,
    )


def _make_cnt(interpret=False):
    scratch = [
        pltpu.VMEM((GB, CB), jnp.int32),
        pltpu.VMEM((CB, F), jnp.float32),
        pltpu.VMEM_SHARED((NPAD, F), jnp.float32),
    ]

    def body(dsts_hbm, cnt_hbm, dst_v, one_v, cacc):
        cid = lax.axis_index("c")
        sid = lax.axis_index("s")
        wid = sid * NC + cid

        _fill_f32(one_v, CB, F, 0.0)
        for i in range(RPT // CB):
            pltpu.sync_copy(one_v, cacc.at[pl.ds(sid * RPT + i * CB, CB)])
        _fill_f32(one_v, CB, F, 1.0)
        plsc.subcore_barrier()

        def gbody(g, carry):
            pltpu.sync_copy(dsts_hbm.at[pl.ds(wid * CPW + g * GB, GB)], dst_v)

            def ebody(k, c2):
                pltpu.sync_copy(one_v, cacc.at[dst_v.at[k]], add=True)
                return c2

            lax.fori_loop(0, GB, ebody, 0)
            return carry

        lax.fori_loop(0, CPW // GB, gbody, 0)
        plsc.subcore_barrier()

        for i in range(RPT // CB):
            r0 = sid * RPT + i * CB
            pltpu.sync_copy(cacc.at[pl.ds(r0, CB)], one_v)
            pltpu.sync_copy(one_v, cnt_hbm.at[pl.ds(cid * NPAD + r0, CB)])

    return pl.kernel(
        body,
        out_type=jax.ShapeDtypeStruct((NC * NPAD, F), jnp.float32),
        mesh=_mesh(),
        scratch_types=scratch,
        interpret=interpret,
    )


def _lin_body(relu, p_ref, h_ref, c_ref, w_ref, b_ref, o_ref):
    s = p_ref[0] + p_ref[1] + h_ref[...]
    cnt = c_ref[0, :, 0:1] + c_ref[1, :, 0:1] + 1.0
    m = s / cnt
    y = jnp.dot(m, w_ref[...], preferred_element_type=jnp.float32) + b_ref[...]
    o_ref[...] = jnp.maximum(y, 0.0) if relu else y


def _make_lin(out_ch, relu, interpret=False):
    R = 1000
    grid = (N // R,)
    return pl.pallas_call(
        functools.partial(_lin_body, relu),
        grid=grid,
        in_specs=[
            pl.BlockSpec((2, R, F), lambda i: (0, i, 0)),
            pl.BlockSpec((R, F), lambda i: (i, 0)),
            pl.BlockSpec((2, R, L), lambda i: (0, i, 0)),
            pl.BlockSpec((F, out_ch), lambda i: (0, 0)),
            pl.BlockSpec((1, out_ch), lambda i: (0, 0)),
        ],
        out_specs=pl.BlockSpec((R, out_ch), lambda i: (i, 0)),
        out_shape=jax.ShapeDtypeStruct((N, out_ch), jnp.float32),
        interpret=interpret,
    )


def _build(interpret=False):
    seg = _make_seg(interpret)
    cntk = _make_cnt(interpret)
    lin_h1 = _make_lin(F, True, interpret)
    lin_h2 = _make_lin(F, True, interpret)
    lin_out = _make_lin(40, False, interpret)

    def run(x, adj, W1, u1, c1, b1, W2, u2, c2, b2, W3, u3, c3, b3):
        del u1, c1, u2, c2, u3, c3
        src = adj[0].astype(jnp.int32)
        dst = adj[1].astype(jnp.int32)
        pad = EPAD - E
        srcs = jnp.concatenate([src, jnp.zeros((pad,), jnp.int32)])
        trash = N + (jnp.arange(pad, dtype=jnp.int32) % (NPAD - N))
        dsts = jnp.concatenate([dst, trash])
        srcs = srcs.reshape(NW * CPW, CB)
        dsts = dsts.reshape(NW * CPW, CB)

        cnt = cntk(dsts)
        cr = cnt.reshape(NC, NPAD, F)[:, :, :L]
        p1 = seg(x, srcs, dsts)
        h1 = lin_h1(p1.reshape(NC, NPAD, F), x, cr, W1, b1.reshape(1, F))
        p2 = seg(h1, srcs, dsts)
        h2 = lin_h2(p2.reshape(NC, NPAD, F), h1, cr, W2, b2.reshape(1, F))
        p3 = seg(h2, srcs, dsts)
        out = lin_out(p3.reshape(NC, NPAD, F), h2, cr, W3, b3.reshape(1, 40))
        return out

    return run


_RUN = None


def kernel(x, adj, W1, u1, c1, b1, W2, u2, c2, b2, W3, u3, c3, b3):
    global _RUN
    if _RUN is None:
        _RUN = _build()
    return _RUN(x, adj, W1, u1, c1, b1, W2, u2, c2, b2, W3, u3, c3, b3)


# speedup vs baseline: 1.4928x; 1.4928x over previous
import functools

import jax
import jax.numpy as jnp
from jax import lax
from jax.experimental import pallas as pl
from jax.experimental.pallas import tpu as pltpu
from jax.experimental.pallas import tpu_sc as plsc

N = 10000
F = 128
E = 320000
NC = 2
NS = 16
NW = NC * NS
L = 16
CB = 128
CPW = 80
CPW_FAST = 152
CPW_SLOW = 8
GB = 8
EW = CPW * CB
EPAD = EW * NW
NPAD = 10240
RPT = NPAD // NS


def _mesh():
    return plsc.VectorSubcoreMesh(
        core_axis_name="c", subcore_axis_name="s", num_cores=NC, num_subcores=NS
    )


def _fill_f32(ref, nrows, ncols, value):
    v = jnp.full((L,), value, jnp.float32)

    def body(i, carry):
        for j in range(ncols // L):
            ref[i, pl.ds(j * L, L)] = v
        return carry

    lax.fori_loop(0, nrows, body, 0)


def _make_seg(interpret=False):
    scratch = [
        pltpu.VMEM((GB, CB), jnp.int32),
        pltpu.VMEM((GB, CB), jnp.int32),
        pltpu.VMEM((CB, F), jnp.float32),
        pltpu.VMEM((CB, F), jnp.float32),
        pltpu.VMEM_SHARED((NPAD, F), jnp.float32),
        pltpu.SemaphoreType.DMA,
        pltpu.SemaphoreType.DMA,
        pltpu.SemaphoreType.DMA,
        pltpu.SemaphoreType.DMA,
    ]

    def body(h_hbm, srcs_hbm, dsts_hbm, out_hbm,
             src_v, dst_v, rows_a, rows_b, acc, sem_a, sem_b, sem_sa, sem_sb):
        cid = lax.axis_index("c")
        sid = lax.axis_index("s")
        base = jnp.where(cid == 1, sid * CPW_FAST,
                         NS * CPW_FAST + sid * CPW_SLOW)
        ng = jnp.where(cid == 1, CPW_FAST // GB, CPW_SLOW // GB)

        _fill_f32(rows_a, CB, F, 0.0)
        for i in range(RPT // CB):
            pltpu.sync_copy(rows_a, acc.at[pl.ds(sid * RPT + i * CB, CB)])
        plsc.subcore_barrier()

        bufs = ((rows_a, sem_a, sem_sa), (rows_b, sem_b, sem_sb))

        HCB = CB // 2

        def gath(k, buf, s1, s2):
            c1 = pltpu.async_copy(h_hbm.at[src_v.at[k, pl.ds(0, HCB)]],
                                  buf.at[pl.ds(0, HCB)], s1)
            c2 = pltpu.async_copy(h_hbm.at[src_v.at[k, pl.ds(HCB, HCB)]],
                                  buf.at[pl.ds(HCB, HCB)], s2)
            return (c1, c2)

        def gbody(g, carry):
            pltpu.sync_copy(srcs_hbm.at[pl.ds(base + g * GB, GB)], src_v)
            pltpu.sync_copy(dsts_hbm.at[pl.ds(base + g * GB, GB)], dst_v)
            cpg = [gath(0, rows_a, sem_a, sem_sa),
                   gath(1, rows_b, sem_b, sem_sb)]
            for k in range(GB):
                r, s1, s2 = bufs[k % 2]
                for c in cpg[k % 2]:
                    c.wait()
                pltpu.sync_copy(r, acc.at[dst_v.at[k]], add=True)
                if k + 2 < GB:
                    cpg[k % 2] = gath(k + 2, r, s1, s2)
            return carry

        lax.fori_loop(0, ng, gbody, 0)
        plsc.subcore_barrier()

        for i in range(RPT // CB):
            r0 = sid * RPT + i * CB
            pltpu.sync_copy(acc.at[pl.ds(r0, CB)], rows_a)
            pltpu.sync_copy(rows_a, out_hbm.at[pl.ds(cid * NPAD + r0, CB)])

    return pl.kernel(
        body,
        out_type=jax.ShapeDtypeStruct((NC * NPAD, F), jnp.float32),
        mesh=_mesh(),
        scratch_types=scratch,
        interpret=interpret,
    )


def _make_cnt(interpret=False):
    scratch = [
        pltpu.VMEM((GB, CB), jnp.int32),
        pltpu.VMEM((CB, F), jnp.float32),
        pltpu.VMEM_SHARED((NPAD, F), jnp.float32),
    ]

    def body(dsts_hbm, cnt_hbm, dst_v, one_v, cacc):
        cid = lax.axis_index("c")
        sid = lax.axis_index("s")
        wid = sid * NC + cid

        _fill_f32(one_v, CB, F, 0.0)
        for i in range(RPT // CB):
            pltpu.sync_copy(one_v, cacc.at[pl.ds(sid * RPT + i * CB, CB)])
        _fill_f32(one_v, CB, F, 1.0)
        plsc.subcore_barrier()

        def gbody(g, carry):
            pltpu.sync_copy(dsts_hbm.at[pl.ds(wid * CPW + g * GB, GB)], dst_v)

            def ebody(k, c2):
                pltpu.sync_copy(one_v, cacc.at[dst_v.at[k]], add=True)
                return c2

            lax.fori_loop(0, GB, ebody, 0)
            return carry

        lax.fori_loop(0, CPW // GB, gbody, 0)
        plsc.subcore_barrier()

        for i in range(RPT // CB):
            r0 = sid * RPT + i * CB
            pltpu.sync_copy(cacc.at[pl.ds(r0, CB)], one_v)
            pltpu.sync_copy(one_v, cnt_hbm.at[pl.ds(cid * NPAD + r0, CB)])

    return pl.kernel(
        body,
        out_type=jax.ShapeDtypeStruct((NC * NPAD, F), jnp.float32),
        mesh=_mesh(),
        scratch_types=scratch,
        interpret=interpret,
    )


def _lin_body(relu, p_ref, h_ref, c_ref, w_ref, b_ref, o_ref):
    s = p_ref[0] + p_ref[1] + h_ref[...]
    cnt = c_ref[0, :, 0:1] + c_ref[1, :, 0:1] + 1.0
    m = s / cnt
    y = jnp.dot(m, w_ref[...], preferred_element_type=jnp.float32) + b_ref[...]
    o_ref[...] = jnp.maximum(y, 0.0) if relu else y


def _make_lin(out_ch, relu, interpret=False):
    R = 1000
    grid = (N // R,)
    return pl.pallas_call(
        functools.partial(_lin_body, relu),
        grid=grid,
        in_specs=[
            pl.BlockSpec((2, R, F), lambda i: (0, i, 0)),
            pl.BlockSpec((R, F), lambda i: (i, 0)),
            pl.BlockSpec((2, R, L), lambda i: (0, i, 0)),
            pl.BlockSpec((F, out_ch), lambda i: (0, 0)),
            pl.BlockSpec((1, out_ch), lambda i: (0, 0)),
        ],
        out_specs=pl.BlockSpec((R, out_ch), lambda i: (i, 0)),
        out_shape=jax.ShapeDtypeStruct((N, out_ch), jnp.float32),
        interpret=interpret,
    )


def _build(interpret=False):
    seg = _make_seg(interpret)
    cntk = _make_cnt(interpret)
    lin_h1 = _make_lin(F, True, interpret)
    lin_h2 = _make_lin(F, True, interpret)
    lin_out = _make_lin(40, False, interpret)

    def run(x, adj, W1, u1, c1, b1, W2, u2, c2, b2, W3, u3, c3, b3):
        del u1, c1, u2, c2, u3, c3
        src = adj[0].astype(jnp.int32)
        dst = adj[1].astype(jnp.int32)
        pad = EPAD - E
        srcs = jnp.concatenate([src, jnp.zeros((pad,), jnp.int32)])
        trash = N + (jnp.arange(pad, dtype=jnp.int32) % (NPAD - N))
        dsts = jnp.concatenate([dst, trash])
        srcs = srcs.reshape(NW * CPW, CB)
        dsts = dsts.reshape(NW * CPW, CB)

        cnt = cntk(dsts)
        cr = cnt.reshape(NC, NPAD, F)[:, :, :L]
        p1 = seg(x, srcs, dsts)
        h1 = lin_h1(p1.reshape(NC, NPAD, F), x, cr, W1, b1.reshape(1, F))
        p2 = seg(h1, srcs, dsts)
        h2 = lin_h2(p2.reshape(NC, NPAD, F), h1, cr, W2, b2.reshape(1, F))
        p3 = seg(h2, srcs, dsts)
        out = lin_out(p3.reshape(NC, NPAD, F), h2, cr, W3, b3.reshape(1, 40))
        return out

    return run


_RUN = None


def kernel(x, adj, W1, u1, c1, b1, W2, u2, c2, b2, W3, u3, c3, b3):
    global _RUN
    if _RUN is None:
        _RUN = _build()
    return _RUN(x, adj, W1, u1, c1, b1, W2, u2, c2, b2, W3, u3, c3, b3)
